# Initial kernel scaffold; baseline (speedup 1.0000x reference)
import jax
import jax.numpy as jnp
from jax.experimental import pallas as pl


def kernel(x, W):
    raise NotImplementedError("write your pallas kernel here")


# speedup vs baseline: 2.2227x; 2.2227x over previous
import functools

import jax
import jax.numpy as jnp
from jax import lax
from jax.experimental import pallas as pl
from jax.experimental.pallas import tpu as pltpu
from jax.experimental.pallas import tpu_sc as plsc

_INFO = plsc.get_sparse_core_info()
_NC = _INFO.num_cores
_NS = _INFO.num_subcores
_NW = _NC * _NS

_D = 768
_B = 4 * 8192
_PER_W = _B // _NW
_CHUNK = 64
_NCHUNK = _PER_W // _CHUNK


def _sc_gather(xf, W):
    mesh = plsc.VectorSubcoreMesh(core_axis_name="c", subcore_axis_name="s")

    @functools.partial(
        pl.kernel,
        out_type=jax.ShapeDtypeStruct((_B, _D), jnp.float32),
        mesh=mesh,
        scratch_types=[
            pltpu.VMEM((_NCHUNK, _CHUNK), jnp.int32),
            pltpu.VMEM((_CHUNK, _D), jnp.float32),
            pltpu.SemaphoreType.DMA,
        ],
    )
    def k(x_hbm, w_hbm, out_hbm, idx_v, rows_v, gsem):
        wid = lax.axis_index("s") * _NC + lax.axis_index("c")
        base = wid * _PER_W
        pltpu.sync_copy(x_hbm.at[wid], idx_v)
        for g in range(_NCHUNK):
            pltpu.async_copy(w_hbm.at[idx_v.at[g]], rows_v, gsem).wait()
            pltpu.sync_copy(rows_v, out_hbm.at[pl.ds(base + g * _CHUNK, _CHUNK)])

    return k(xf, W)


def kernel(x, W):
    xf = x.reshape(_NW, _NCHUNK, _CHUNK).astype(jnp.int32)
    out = _sc_gather(xf, W)
    return out.reshape(x.shape[0], x.shape[1], _D)


# speedup vs baseline: 2.4838x; 1.1174x over previous
import functools

import jax
import jax.numpy as jnp
from jax import lax
from jax.experimental import pallas as pl
from jax.experimental.pallas import tpu as pltpu
from jax.experimental.pallas import tpu_sc as plsc

_INFO = plsc.get_sparse_core_info()
_NC = _INFO.num_cores
_NS = _INFO.num_subcores
_NW = _NC * _NS

_D = 768
_B = 4 * 8192
_PER_W = _B // _NW
_CHUNK = 64
_NCHUNK = _PER_W // _CHUNK


def _sc_gather(xf, W):
    mesh = plsc.VectorSubcoreMesh(core_axis_name="c", subcore_axis_name="s")

    @functools.partial(
        pl.kernel,
        out_type=jax.ShapeDtypeStruct((_B, _D), jnp.float32),
        mesh=mesh,
        scratch_types=[
            pltpu.VMEM((_NCHUNK, _CHUNK), jnp.int32),
            pltpu.VMEM((2, _CHUNK, _D), jnp.float32),
            pltpu.SemaphoreType.DMA,
            pltpu.SemaphoreType.DMA,
        ],
    )
    def k(x_hbm, w_hbm, out_hbm, idx_v, rows_v, gsem, ssem):
        wid = lax.axis_index("s") * _NC + lax.axis_index("c")
        base = wid * _PER_W
        pltpu.sync_copy(x_hbm.at[wid], idx_v)

        def gather(g):
            return pltpu.async_copy(
                w_hbm.at[idx_v.at[g]], rows_v.at[g % 2], gsem)

        def scatter(g):
            return pltpu.async_copy(
                rows_v.at[g % 2],
                out_hbm.at[pl.ds(base + g * _CHUNK, _CHUNK)], ssem)

        gathers = [gather(0)]
        scatters = []
        for g in range(_NCHUNK):
            if g >= 1:
                scatters[g - 1].wait()
            if g + 1 < _NCHUNK:
                gathers.append(gather(g + 1))
            gathers[g].wait()
            scatters.append(scatter(g))
        scatters[-1].wait()

    return k(xf, W)


def kernel(x, W):
    xf = x.reshape(_NW, _NCHUNK, _CHUNK).astype(jnp.int32)
    out = _sc_gather(xf, W)
    return out.reshape(x.shape[0], x.shape[1], _D)
